# baseline (speedup 1.0000x reference)
import functools

import jax
import jax.numpy as jnp
from jax import lax
from jax.experimental import pallas as pl
from jax.experimental.pallas import tpu as pltpu
from jax.experimental.pallas import tpu_sc as plsc

_L = 16
_NC = 2
_NS = 16
_NW = _NC * _NS
_G = 4


def _nn_body(n, qpw, x1_hbm, x2_hbm, minsq_hbm, arg_hbm,
             kx_v, ky_v, kz_v, qx_v, qy_v, qz_v, om_v, oa_v):
    tiles_per_b = n // qpw
    nchunks = n // _L
    wid = lax.axis_index("s") * _NC + lax.axis_index("c")
    b = wid // tiles_per_b
    qbase = (wid % tiles_per_b) * qpw

    kofs = b * (3 * n)
    qofs = kofs + qbase
    pltpu.sync_copy(x2_hbm.at[pl.ds(kofs, n)], kx_v)
    pltpu.sync_copy(x2_hbm.at[pl.ds(kofs + n, n)], ky_v)
    pltpu.sync_copy(x2_hbm.at[pl.ds(kofs + 2 * n, n)], kz_v)
    pltpu.sync_copy(x1_hbm.at[pl.ds(qofs, qpw)], qx_v)
    pltpu.sync_copy(x1_hbm.at[pl.ds(qofs + n, qpw)], qy_v)
    pltpu.sync_copy(x1_hbm.at[pl.ds(qofs + 2 * n, qpw)], qz_v)

    def block(blk, _):
        qb = blk * (_L * _G)
        qx = [qx_v[pl.ds(qb + i * _L, _L)] for i in range(_G)]
        qy = [qy_v[pl.ds(qb + i * _L, _L)] for i in range(_G)]
        qz = [qz_v[pl.ds(qb + i * _L, _L)] for i in range(_G)]

        def scan_chunk(c, carry):
            run_min, carg, cuarg = carry
            base = c * _L
            kxc = kx_v[pl.ds(base, _L)]
            kyc = ky_v[pl.ds(base, _L)]
            kzc = kz_v[pl.ds(base, _L)]
            cm = [None] * _G
            cpos = [None] * _G
            for u in range(_L):
                sel = jnp.full((_L,), u, jnp.int32)
                kxu = kxc.at[sel].get(mode="promise_in_bounds")
                kyu = kyc.at[sel].get(mode="promise_in_bounds")
                kzu = kzc.at[sel].get(mode="promise_in_bounds")
                uvec = jnp.full((_L,), u, jnp.int32)
                for i in range(_G):
                    dx = qx[i] - kxu
                    dy = qy[i] - kyu
                    dz = qz[i] - kzu
                    d2 = dx * dx + dy * dy + dz * dz
                    if cm[i] is None:
                        cm[i] = d2
                        cpos[i] = uvec
                    else:
                        upd = d2 < cm[i]
                        cm[i] = jnp.minimum(cm[i], d2)
                        cpos[i] = jnp.where(upd, uvec, cpos[i])
            cvec = jnp.full((_L,), c, jnp.int32)
            new_min, new_carg, new_cuarg = [], [], []
            for i in range(_G):
                upd = cm[i] < run_min[i]
                new_min.append(jnp.minimum(run_min[i], cm[i]))
                new_carg.append(jnp.where(upd, cvec, carg[i]))
                new_cuarg.append(jnp.where(upd, cpos[i], cuarg[i]))
            return tuple(new_min), tuple(new_carg), tuple(new_cuarg)

        init = (tuple(jnp.full((_L,), jnp.inf, jnp.float32)
                      for _i in range(_G)),
                tuple(jnp.zeros((_L,), jnp.int32) for _i in range(_G)),
                tuple(jnp.zeros((_L,), jnp.int32) for _i in range(_G)))
        run_min, carg, cuarg = lax.fori_loop(0, nchunks, scan_chunk, init)

        for i in range(_G):
            om_v[pl.ds(qb + i * _L, _L)] = run_min[i]
            oa_v[pl.ds(qb + i * _L, _L)] = carg[i] * _L + cuarg[i]
        return _

    lax.fori_loop(0, qpw // (_L * _G), block, 0)

    obase = b * n + qbase
    pltpu.sync_copy(om_v, minsq_hbm.at[pl.ds(obase, qpw)])
    pltpu.sync_copy(oa_v, arg_hbm.at[pl.ds(obase, qpw)])


@functools.partial(jax.jit, static_argnums=(2, 3))
def _nn_sc(x1t, x2t, B, n):
    qpw = (B * n) // _NW
    mesh = plsc.VectorSubcoreMesh(core_axis_name="c", subcore_axis_name="s")
    body = functools.partial(_nn_body, n, qpw)
    ker = pl.kernel(
        body,
        out_type=[jax.ShapeDtypeStruct((B * n,), jnp.float32),
                  jax.ShapeDtypeStruct((B * n,), jnp.int32)],
        mesh=mesh,
        scratch_types=[
            pltpu.VMEM((n,), jnp.float32),
            pltpu.VMEM((n,), jnp.float32),
            pltpu.VMEM((n,), jnp.float32),
            pltpu.VMEM((qpw,), jnp.float32),
            pltpu.VMEM((qpw,), jnp.float32),
            pltpu.VMEM((qpw,), jnp.float32),
            pltpu.VMEM((qpw,), jnp.float32),
            pltpu.VMEM((qpw,), jnp.int32),
        ],
    )
    minsq, arg = ker(x1t, x2t)
    return minsq.reshape(B, n), arg.reshape(B, n)


def kernel(input1, input2, eps, iters):
    B, n, _ = input1.shape
    x1t = jnp.transpose(input1, (0, 2, 1)).reshape(-1)
    x2t = jnp.transpose(input2, (0, 2, 1)).reshape(-1)
    minsq, arg = _nn_sc(x1t, x2t, B, n)

    m = jnp.sqrt(minsq)

    def body(_, carry):
        price, _mc = carry
        mc = m - price
        return price + eps * mc, mc

    _price, mc = lax.fori_loop(
        0, iters, body, (jnp.zeros_like(m), jnp.zeros_like(m)))
    arg = jnp.where(iters >= 1, arg, jnp.zeros_like(arg))
    return jnp.sqrt(mc), arg


# speedup vs baseline: 1.0123x; 1.0123x over previous
import functools

import jax
import jax.numpy as jnp
from jax import lax
from jax.experimental import pallas as pl
from jax.experimental.pallas import tpu as pltpu
from jax.experimental.pallas import tpu_sc as plsc

_L = 16
_NC = 2
_NS = 16
_NW = _NC * _NS
_G = 1


def _nn_body(n, qpw, x1_hbm, x2_hbm, minsq_hbm, arg_hbm,
             kx_v, ky_v, kz_v, qx_v, qy_v, qz_v, om_v, oa_v):
    tiles_per_b = n // qpw
    nchunks = n // _L
    wid = lax.axis_index("s") * _NC + lax.axis_index("c")
    b = wid // tiles_per_b
    qbase = (wid % tiles_per_b) * qpw

    kofs = b * (3 * n)
    qofs = kofs + qbase
    pltpu.sync_copy(x2_hbm.at[pl.ds(kofs, n)], kx_v)
    pltpu.sync_copy(x2_hbm.at[pl.ds(kofs + n, n)], ky_v)
    pltpu.sync_copy(x2_hbm.at[pl.ds(kofs + 2 * n, n)], kz_v)
    pltpu.sync_copy(x1_hbm.at[pl.ds(qofs, qpw)], qx_v)
    pltpu.sync_copy(x1_hbm.at[pl.ds(qofs + n, qpw)], qy_v)
    pltpu.sync_copy(x1_hbm.at[pl.ds(qofs + 2 * n, qpw)], qz_v)

    def block(blk, _):
        qb = blk * (_L * _G)
        qx = [qx_v[pl.ds(qb + i * _L, _L)] for i in range(_G)]
        qy = [qy_v[pl.ds(qb + i * _L, _L)] for i in range(_G)]
        qz = [qz_v[pl.ds(qb + i * _L, _L)] for i in range(_G)]

        def scan_chunk(c, carry):
            run_min, carg, cuarg = carry
            base = c * _L
            kxc = kx_v[pl.ds(base, _L)]
            kyc = ky_v[pl.ds(base, _L)]
            kzc = kz_v[pl.ds(base, _L)]
            cm = [None] * _G
            cpos = [None] * _G
            for u in range(_L):
                sel = jnp.full((_L,), u, jnp.int32)
                kxu = kxc.at[sel].get(mode="promise_in_bounds")
                kyu = kyc.at[sel].get(mode="promise_in_bounds")
                kzu = kzc.at[sel].get(mode="promise_in_bounds")
                uvec = jnp.full((_L,), u, jnp.int32)
                for i in range(_G):
                    dx = qx[i] - kxu
                    dy = qy[i] - kyu
                    dz = qz[i] - kzu
                    d2 = dx * dx + dy * dy + dz * dz
                    if cm[i] is None:
                        cm[i] = d2
                        cpos[i] = uvec
                    else:
                        upd = d2 < cm[i]
                        cm[i] = jnp.minimum(cm[i], d2)
                        cpos[i] = jnp.where(upd, uvec, cpos[i])
            cvec = jnp.full((_L,), c, jnp.int32)
            new_min, new_carg, new_cuarg = [], [], []
            for i in range(_G):
                upd = cm[i] < run_min[i]
                new_min.append(jnp.minimum(run_min[i], cm[i]))
                new_carg.append(jnp.where(upd, cvec, carg[i]))
                new_cuarg.append(jnp.where(upd, cpos[i], cuarg[i]))
            return tuple(new_min), tuple(new_carg), tuple(new_cuarg)

        init = (tuple(jnp.full((_L,), jnp.inf, jnp.float32)
                      for _i in range(_G)),
                tuple(jnp.zeros((_L,), jnp.int32) for _i in range(_G)),
                tuple(jnp.zeros((_L,), jnp.int32) for _i in range(_G)))

        run_min, carg, cuarg = lax.fori_loop(0, nchunks, scan_chunk, init)

        for i in range(_G):
            om_v[pl.ds(qb + i * _L, _L)] = run_min[i]
            oa_v[pl.ds(qb + i * _L, _L)] = carg[i] * _L + cuarg[i]
        return _

    lax.fori_loop(0, qpw // (_L * _G), block, 0)

    obase = b * n + qbase
    pltpu.sync_copy(om_v, minsq_hbm.at[pl.ds(obase, qpw)])
    pltpu.sync_copy(oa_v, arg_hbm.at[pl.ds(obase, qpw)])


@functools.partial(jax.jit, static_argnums=(2, 3))
def _nn_sc(x1t, x2t, B, n):
    qpw = (B * n) // _NW
    mesh = plsc.VectorSubcoreMesh(core_axis_name="c", subcore_axis_name="s")
    body = functools.partial(_nn_body, n, qpw)
    ker = pl.kernel(
        body,
        out_type=[jax.ShapeDtypeStruct((B * n,), jnp.float32),
                  jax.ShapeDtypeStruct((B * n,), jnp.int32)],
        mesh=mesh,
        scratch_types=[
            pltpu.VMEM((n,), jnp.float32),
            pltpu.VMEM((n,), jnp.float32),
            pltpu.VMEM((n,), jnp.float32),
            pltpu.VMEM((qpw,), jnp.float32),
            pltpu.VMEM((qpw,), jnp.float32),
            pltpu.VMEM((qpw,), jnp.float32),
            pltpu.VMEM((qpw,), jnp.float32),
            pltpu.VMEM((qpw,), jnp.int32),
        ],
    )
    minsq, arg = ker(x1t, x2t)
    return minsq.reshape(B, n), arg.reshape(B, n)


def kernel(input1, input2, eps, iters):
    B, n, _ = input1.shape
    x1t = jnp.transpose(input1, (0, 2, 1)).reshape(-1)
    x2t = jnp.transpose(input2, (0, 2, 1)).reshape(-1)
    minsq, arg = _nn_sc(x1t, x2t, B, n)

    m = jnp.sqrt(minsq)

    def body(_, carry):
        price, _mc = carry
        mc = m - price
        return price + eps * mc, mc

    _price, mc = lax.fori_loop(
        0, iters, body, (jnp.zeros_like(m), jnp.zeros_like(m)))
    arg = jnp.where(iters >= 1, arg, jnp.zeros_like(arg))
    return jnp.sqrt(mc), arg


# speedup vs baseline: 1.0145x; 1.0021x over previous
import functools

import jax
import jax.numpy as jnp
from jax import lax
from jax.experimental import pallas as pl
from jax.experimental.pallas import tpu as pltpu
from jax.experimental.pallas import tpu_sc as plsc

_L = 16
_NC = 2
_NS = 16
_NW = _NC * _NS
_G = 1


def _nn_body(n, qpw, x1_hbm, x2_hbm, minsq_hbm, arg_hbm,
             kx_v, ky_v, kz_v, qx_v, qy_v, qz_v, om_v, oa_v):
    tiles_per_b = n // qpw
    nchunks = n // _L
    wid = lax.axis_index("s") * _NC + lax.axis_index("c")
    b = wid // tiles_per_b
    qbase = (wid % tiles_per_b) * qpw

    kofs = b * (3 * n)
    qofs = kofs + qbase
    pltpu.sync_copy(x2_hbm.at[pl.ds(kofs, n)], kx_v)
    pltpu.sync_copy(x2_hbm.at[pl.ds(kofs + n, n)], ky_v)
    pltpu.sync_copy(x2_hbm.at[pl.ds(kofs + 2 * n, n)], kz_v)
    pltpu.sync_copy(x1_hbm.at[pl.ds(qofs, qpw)], qx_v)
    pltpu.sync_copy(x1_hbm.at[pl.ds(qofs + n, qpw)], qy_v)
    pltpu.sync_copy(x1_hbm.at[pl.ds(qofs + 2 * n, qpw)], qz_v)

    def block(blk, _):
        qb = blk * (_L * _G)
        qx = [qx_v[pl.ds(qb + i * _L, _L)] for i in range(_G)]
        qy = [qy_v[pl.ds(qb + i * _L, _L)] for i in range(_G)]
        qz = [qz_v[pl.ds(qb + i * _L, _L)] for i in range(_G)]

        def scan_chunk(c, carry):
            run_min, carg, cuarg = carry
            base = c * _L
            kxc = kx_v[pl.ds(base, _L)]
            kyc = ky_v[pl.ds(base, _L)]
            kzc = kz_v[pl.ds(base, _L)]
            cm = [None] * _G
            cpos = [None] * _G
            for u in range(_L):
                sel = jnp.full((_L,), u, jnp.int32)
                kxu = kxc.at[sel].get(mode="promise_in_bounds")
                kyu = kyc.at[sel].get(mode="promise_in_bounds")
                kzu = kzc.at[sel].get(mode="promise_in_bounds")
                uvec = jnp.full((_L,), u, jnp.int32)
                for i in range(_G):
                    dx = qx[i] - kxu
                    dy = qy[i] - kyu
                    dz = qz[i] - kzu
                    d2 = dx * dx + dy * dy + dz * dz
                    if cm[i] is None:
                        cm[i] = d2
                        cpos[i] = uvec
                    else:
                        upd = d2 < cm[i]
                        cm[i] = jnp.minimum(cm[i], d2)
                        cpos[i] = jnp.where(upd, uvec, cpos[i])
            cvec = jnp.full((_L,), c, jnp.int32)
            new_min, new_carg, new_cuarg = [], [], []
            for i in range(_G):
                upd = cm[i] < run_min[i]
                new_min.append(jnp.minimum(run_min[i], cm[i]))
                new_carg.append(jnp.where(upd, cvec, carg[i]))
                new_cuarg.append(jnp.where(upd, cpos[i], cuarg[i]))
            return tuple(new_min), tuple(new_carg), tuple(new_cuarg)

        init = (tuple(jnp.full((_L,), jnp.inf, jnp.float32)
                      for _i in range(_G)),
                tuple(jnp.zeros((_L,), jnp.int32) for _i in range(_G)),
                tuple(jnp.zeros((_L,), jnp.int32) for _i in range(_G)))

        run_min, carg, cuarg = plsc.parallel_loop(
            0, nchunks, carry=init, unroll=2)(scan_chunk)

        for i in range(_G):
            om_v[pl.ds(qb + i * _L, _L)] = run_min[i]
            oa_v[pl.ds(qb + i * _L, _L)] = carg[i] * _L + cuarg[i]
        return _

    lax.fori_loop(0, qpw // (_L * _G), block, 0)

    obase = b * n + qbase
    pltpu.sync_copy(om_v, minsq_hbm.at[pl.ds(obase, qpw)])
    pltpu.sync_copy(oa_v, arg_hbm.at[pl.ds(obase, qpw)])


@functools.partial(jax.jit, static_argnums=(2, 3))
def _nn_sc(x1t, x2t, B, n):
    qpw = (B * n) // _NW
    mesh = plsc.VectorSubcoreMesh(core_axis_name="c", subcore_axis_name="s")
    body = functools.partial(_nn_body, n, qpw)
    ker = pl.kernel(
        body,
        out_type=[jax.ShapeDtypeStruct((B * n,), jnp.float32),
                  jax.ShapeDtypeStruct((B * n,), jnp.int32)],
        mesh=mesh,
        scratch_types=[
            pltpu.VMEM((n,), jnp.float32),
            pltpu.VMEM((n,), jnp.float32),
            pltpu.VMEM((n,), jnp.float32),
            pltpu.VMEM((qpw,), jnp.float32),
            pltpu.VMEM((qpw,), jnp.float32),
            pltpu.VMEM((qpw,), jnp.float32),
            pltpu.VMEM((qpw,), jnp.float32),
            pltpu.VMEM((qpw,), jnp.int32),
        ],
    )
    minsq, arg = ker(x1t, x2t)
    return minsq.reshape(B, n), arg.reshape(B, n)


def kernel(input1, input2, eps, iters):
    B, n, _ = input1.shape
    x1t = jnp.transpose(input1, (0, 2, 1)).reshape(-1)
    x2t = jnp.transpose(input2, (0, 2, 1)).reshape(-1)
    minsq, arg = _nn_sc(x1t, x2t, B, n)

    m = jnp.sqrt(minsq)

    def body(_, carry):
        price, _mc = carry
        mc = m - price
        return price + eps * mc, mc

    _price, mc = lax.fori_loop(
        0, iters, body, (jnp.zeros_like(m), jnp.zeros_like(m)))
    arg = jnp.where(iters >= 1, arg, jnp.zeros_like(arg))
    return jnp.sqrt(mc), arg
